# baseline (speedup 1.0000x reference)
import math

import numpy as np
import jax
import jax.numpy as jnp
from jax.experimental import pallas as pl

B = 2
P = 2048
D = 1024
H = 16
DK = 64
FF = 32
HOR = 24
NL = 2
L = D
U = 6
SCALE = float(DK ** 0.5)


def _pe_np():
    pe = np.zeros((L, D), np.float32)
    pos = np.arange(L, dtype=np.float32)[:, None]
    div = np.exp(np.arange(0, D, 2, dtype=np.float32) * (-math.log(10000.0) / D))
    pe[:, 0::2] = np.sin(pos * div)
    pe[:, 1::2] = np.cos(pos * div)
    return pe


_PE = _pe_np()

_RB = 256


def _embed_body(x_ref, w_ref, b_ref, pe_ref, o_ref):
    xb = x_ref[0]
    wb = w_ref[...]
    acc = jax.lax.dot_general(wb, xb, (((0,), (0,)), ((), ())),
                              preferred_element_type=jnp.float32)
    o_ref[0] = acc + b_ref[...] + pe_ref[...]


def _embed(x, emb_w, emb_b):
    return pl.pallas_call(
        _embed_body,
        grid=(B, L // _RB),
        in_specs=[
            pl.BlockSpec((1, P, D), lambda b, j: (b, 0, 0)),
            pl.BlockSpec((P, _RB), lambda b, j: (0, j)),
            pl.BlockSpec((_RB, 1), lambda b, j: (j, 0)),
            pl.BlockSpec((_RB, D), lambda b, j: (j, 0)),
        ],
        out_specs=pl.BlockSpec((1, _RB, D), lambda b, j: (b, j, 0)),
        out_shape=jax.ShapeDtypeStruct((B, L, D), jnp.float32),
    )(x, emb_w, emb_b.reshape(D, 1), jnp.asarray(_PE))


def _qkv_body(h_ref, qw_ref, kw_ref, vw_ref, qb_ref, kb_ref, vb_ref,
              q_ref, k_ref, v_ref):
    hb = h_ref[0]
    q_ref[0, 0] = jnp.dot(hb, qw_ref[0], preferred_element_type=jnp.float32) + qb_ref[0]
    k_ref[0, 0] = jnp.dot(hb, kw_ref[0], preferred_element_type=jnp.float32) + kb_ref[0]
    v_ref[0, 0] = jnp.dot(hb, vw_ref[0], preferred_element_type=jnp.float32) + vb_ref[0]


def _qkv(h, qwh, kwh, vwh, qbh, kbh, vbh):
    w_spec = pl.BlockSpec((1, D, DK), lambda b, hh: (hh, 0, 0))
    b_spec = pl.BlockSpec((1, 1, DK), lambda b, hh: (hh, 0, 0))
    return pl.pallas_call(
        _qkv_body,
        grid=(B, H),
        in_specs=[pl.BlockSpec((1, L, D), lambda b, hh: (b, 0, 0)),
                  w_spec, w_spec, w_spec, b_spec, b_spec, b_spec],
        out_specs=[pl.BlockSpec((1, 1, L, DK), lambda b, hh: (b, hh, 0, 0))] * 3,
        out_shape=[jax.ShapeDtypeStruct((B, H, L, DK), jnp.float32)] * 3,
    )(h, qwh, kwh, vwh, qbh, kbh, vbh)


def _attn_body(q_ref, k_ref, v_ref, ow_ref, c_ref, idx_ref):
    q = q_ref[0, 0]
    k = k_ref[0, 0]
    v = v_ref[0, 0]
    s = jax.lax.dot_general(q, k, (((1,), (1,)), ((), ())),
                            preferred_element_type=jnp.float32) / SCALE
    m = jnp.max(s.reshape(8, 128, L), axis=-1)
    lin = (jax.lax.broadcasted_iota(jnp.int32, (8, 128), 0) * 128
           + jax.lax.broadcasted_iota(jnp.int32, (8, 128), 1))
    io_row = jax.lax.broadcasted_iota(jnp.int32, (1, L), 1)
    eye_rows = []
    idxs = []
    for _ in range(U):
        vmax = jnp.max(m)
        idx = jnp.min(jnp.where(m == vmax, lin, jnp.int32(L)))
        eye_rows.append((io_row == idx).astype(jnp.float32))
        idxs.append(idx)
        m = jnp.where(lin == idx, jnp.float32(-jnp.inf), m)
    e = jnp.concatenate(eye_rows, axis=0)
    a = jnp.dot(e, s, preferred_element_type=jnp.float32)
    a = a - jnp.max(a, axis=1, keepdims=True)
    w = jnp.exp(a)
    w = w / jnp.sum(w, axis=1, keepdims=True)
    ctx = jnp.dot(w, v, preferred_element_type=jnp.float32)
    c = jnp.dot(ctx, ow_ref[0], preferred_element_type=jnp.float32)
    c_ref[0, 0] = jnp.concatenate(
        [c, jnp.zeros((8 - U, D), jnp.float32)], axis=0)
    lanes = jax.lax.broadcasted_iota(jnp.int32, (1, 128), 1)
    iv = jnp.zeros((1, 128), jnp.int32)
    for j in range(U):
        iv = jnp.where(lanes == j, idxs[j], iv)
    idx_ref[0, 0] = iv


def _attn(q, k, v, owh):
    qkv_spec = pl.BlockSpec((1, 1, L, DK), lambda b, hh: (b, hh, 0, 0))
    return pl.pallas_call(
        _attn_body,
        grid=(B, H),
        in_specs=[qkv_spec, qkv_spec, qkv_spec,
                  pl.BlockSpec((1, DK, D), lambda b, hh: (hh, 0, 0))],
        out_specs=[pl.BlockSpec((1, 1, 8, D), lambda b, hh: (b, hh, 0, 0)),
                   pl.BlockSpec((1, 1, 1, 128), lambda b, hh: (b, hh, 0, 0))],
        out_shape=[jax.ShapeDtypeStruct((B, H, 8, D), jnp.float32),
                   jax.ShapeDtypeStruct((B, H, 1, 128), jnp.int32)],
    )(q, k, v, owh)


def _post_body(h_ref, c_ref, idx_ref, ob_ref, g1_ref, b1_ref,
               w1_ref, fb1_ref, w2_ref, fb2_ref, g2_ref, b2_ref, o_ref):
    hb = h_ref[0]
    cf = c_ref[0].reshape(H * 8, D)
    idxm = idx_ref[0].reshape(H, 128)
    io_col = jax.lax.broadcasted_iota(jnp.int32, (L, 1), 0)
    cols = [(io_col == idxm[hh:hh + 1, :8]).astype(jnp.float32) for hh in range(H)]
    one_t = jnp.concatenate(cols, axis=1)
    delta = jnp.dot(one_t, cf, preferred_element_type=jnp.float32)
    y = hb + delta + ob_ref[...]
    mu = jnp.mean(y, axis=1, keepdims=True)
    var = jnp.mean((y - mu) ** 2, axis=1, keepdims=True)
    hn = (y - mu) / jnp.sqrt(var + 1e-5) * g1_ref[...] + b1_ref[...]
    f = jnp.maximum(
        jnp.dot(hn, w1_ref[...], preferred_element_type=jnp.float32) + fb1_ref[...],
        0.0)
    f = jnp.dot(f, w2_ref[...], preferred_element_type=jnp.float32) + fb2_ref[...]
    z = hn + f
    mu2 = jnp.mean(z, axis=1, keepdims=True)
    var2 = jnp.mean((z - mu2) ** 2, axis=1, keepdims=True)
    o_ref[0] = (z - mu2) / jnp.sqrt(var2 + 1e-5) * g2_ref[...] + b2_ref[...]


def _post(h, c, idx, ob, g1, b1, w1, fb1, w2, fb2, g2, b2):
    row = pl.BlockSpec((1, D), lambda b: (0, 0))
    return pl.pallas_call(
        _post_body,
        grid=(B,),
        in_specs=[
            pl.BlockSpec((1, L, D), lambda b: (b, 0, 0)),
            pl.BlockSpec((1, H, 8, D), lambda b: (b, 0, 0, 0)),
            pl.BlockSpec((1, H, 1, 128), lambda b: (b, 0, 0, 0)),
            row, row, row,
            pl.BlockSpec((D, FF), lambda b: (0, 0)),
            pl.BlockSpec((1, FF), lambda b: (0, 0)),
            pl.BlockSpec((FF, D), lambda b: (0, 0)),
            row, row, row,
        ],
        out_specs=pl.BlockSpec((1, L, D), lambda b: (b, 0, 0)),
        out_shape=jax.ShapeDtypeStruct((B, L, D), jnp.float32),
    )(h, c, idx, ob, g1, b1, w1, fb1, w2, fb2, g2, b2)


def _final_body(h_ref, w_ref, b_ref, o_ref):
    pooled = jnp.mean(h_ref[0], axis=0, keepdims=True)
    o_ref[0] = jnp.dot(pooled, w_ref[...], preferred_element_type=jnp.float32) + b_ref[...]


def _final(h, fc_w, fc_b):
    return pl.pallas_call(
        _final_body,
        grid=(B,),
        in_specs=[
            pl.BlockSpec((1, L, D), lambda b: (b, 0, 0)),
            pl.BlockSpec((D, HOR), lambda b: (0, 0)),
            pl.BlockSpec((1, HOR), lambda b: (0, 0)),
        ],
        out_specs=pl.BlockSpec((1, 1, HOR), lambda b: (b, 0, 0)),
        out_shape=jax.ShapeDtypeStruct((B, 1, HOR), jnp.float32),
    )(h, fc_w, fc_b.reshape(1, HOR))


def kernel(x, emb_w, emb_b, q_w, q_b, k_w, k_b, v_w, v_b, o_w, o_b,
           ff1_w, ff1_b, ff2_w, ff2_b, n1_g, n1_b, n2_g, n2_b, fc_w, fc_b):
    h = _embed(x, emb_w, emb_b)
    for i in range(NL):
        qwh = q_w[i].reshape(D, H, DK).transpose(1, 0, 2)
        kwh = k_w[i].reshape(D, H, DK).transpose(1, 0, 2)
        vwh = v_w[i].reshape(D, H, DK).transpose(1, 0, 2)
        owh = o_w[i].reshape(H, DK, D)
        qbh = q_b[i].reshape(H, 1, DK)
        kbh = k_b[i].reshape(H, 1, DK)
        vbh = v_b[i].reshape(H, 1, DK)
        q, k, v = _qkv(h, qwh, kwh, vwh, qbh, kbh, vbh)
        c, idx = _attn(q, k, v, owh)
        h = _post(h, c, idx,
                  o_b[i].reshape(1, D),
                  n1_g[i].reshape(1, D), n1_b[i].reshape(1, D),
                  ff1_w[i], ff1_b[i].reshape(1, FF),
                  ff2_w[i], ff2_b[i].reshape(1, D),
                  n2_g[i].reshape(1, D), n2_b[i].reshape(1, D))
    out = _final(h, fc_w, fc_b)
    return out.reshape(B, HOR)


# speedup vs baseline: 1.3180x; 1.3180x over previous
import math

import numpy as np
import jax
import jax.numpy as jnp
from jax.experimental import pallas as pl

B = 2
P = 2048
D = 1024
H = 16
DK = 64
FF = 32
HOR = 24
NL = 2
L = D
U = 6
SCALE = float(DK ** 0.5)


def _pe_np():
    pe = np.zeros((L, D), np.float32)
    pos = np.arange(L, dtype=np.float32)[:, None]
    div = np.exp(np.arange(0, D, 2, dtype=np.float32) * (-math.log(10000.0) / D))
    pe[:, 0::2] = np.sin(pos * div)
    pe[:, 1::2] = np.cos(pos * div)
    return pe


_PE = _pe_np()

_RB = 256


def _embed_body(x_ref, w_ref, b_ref, pe_ref, o_ref):
    xb = x_ref[0]
    wb = w_ref[...]
    acc = jax.lax.dot_general(wb, xb, (((0,), (0,)), ((), ())),
                              preferred_element_type=jnp.float32)
    o_ref[0] = acc + b_ref[...] + pe_ref[...]


def _embed(x, emb_w, emb_b):
    return pl.pallas_call(
        _embed_body,
        grid=(B, L // _RB),
        in_specs=[
            pl.BlockSpec((1, P, D), lambda b, j: (b, 0, 0)),
            pl.BlockSpec((P, _RB), lambda b, j: (0, j)),
            pl.BlockSpec((_RB, 1), lambda b, j: (j, 0)),
            pl.BlockSpec((_RB, D), lambda b, j: (j, 0)),
        ],
        out_specs=pl.BlockSpec((1, _RB, D), lambda b, j: (b, j, 0)),
        out_shape=jax.ShapeDtypeStruct((B, L, D), jnp.float32),
    )(x, emb_w, emb_b.reshape(D, 1), jnp.asarray(_PE))


def _layer_body(h_ref, qw_ref, kw_ref, vw_ref, ow_ref, qb_ref, kb_ref, vb_ref,
                ob_ref, g1_ref, b1_ref, w1_ref, fb1_ref, w2_ref, fb2_ref,
                g2_ref, b2_ref, o_ref):
    hb = h_ref[0]
    q_all = jnp.dot(hb, qw_ref[...], preferred_element_type=jnp.float32) + qb_ref[...]
    k_all = jnp.dot(hb, kw_ref[...], preferred_element_type=jnp.float32) + kb_ref[...]
    io_row = jax.lax.broadcasted_iota(jnp.int32, (1, L), 1)
    io_col = jax.lax.broadcasted_iota(jnp.int32, (L, 1), 0)
    lanes8 = jax.lax.broadcasted_iota(jnp.int32, (1, 8), 1)
    lin = (jax.lax.broadcasted_iota(jnp.int32, (8, 128), 0) * 128
           + jax.lax.broadcasted_iota(jnp.int32, (8, 128), 1))
    one_cols = []
    c_rows = []
    for hh in range(H):
        sl = slice(hh * DK, (hh + 1) * DK)
        q_h = q_all[:, sl]
        k_h = k_all[:, sl]
        s = jax.lax.dot_general(q_h, k_h, (((1,), (1,)), ((), ())),
                                preferred_element_type=jnp.float32) / SCALE
        m = jnp.max(s.reshape(8, 128, L), axis=-1)
        eye_rows = []
        iv8 = jnp.zeros((1, 8), jnp.int32)
        for j in range(U):
            vmax = jnp.max(m)
            idx = jnp.min(jnp.where(m == vmax, lin, jnp.int32(L)))
            eye_rows.append((io_row == idx).astype(jnp.float32))
            iv8 = jnp.where(lanes8 == j, idx, iv8)
            m = jnp.where(lin == idx, jnp.float32(-jnp.inf), m)
        e = jnp.concatenate(eye_rows, axis=0)
        q_sel = jnp.dot(e, q_h, preferred_element_type=jnp.float32)
        a = jax.lax.dot_general(q_sel, k_h, (((1,), (1,)), ((), ())),
                                preferred_element_type=jnp.float32) / SCALE
        a = a - jnp.max(a, axis=1, keepdims=True)
        w = jnp.exp(a)
        w = w / jnp.sum(w, axis=1, keepdims=True)
        wh = jnp.dot(w, hb, preferred_element_type=jnp.float32)
        ctx = (jnp.dot(wh, vw_ref[:, sl], preferred_element_type=jnp.float32)
               + vb_ref[:, sl])
        c_h = jnp.dot(ctx, ow_ref[sl, :], preferred_element_type=jnp.float32)
        c_rows.append(jnp.concatenate(
            [c_h, jnp.zeros((8 - U, D), jnp.float32)], axis=0))
        one_cols.append((io_col == iv8).astype(jnp.float32))
    one_t = jnp.concatenate(one_cols, axis=1)
    cf = jnp.concatenate(c_rows, axis=0)
    delta = jnp.dot(one_t, cf, preferred_element_type=jnp.float32)
    y = hb + delta + ob_ref[...]
    mu = jnp.mean(y, axis=1, keepdims=True)
    var = jnp.mean((y - mu) ** 2, axis=1, keepdims=True)
    hn = (y - mu) / jnp.sqrt(var + 1e-5) * g1_ref[...] + b1_ref[...]
    f = jnp.maximum(
        jnp.dot(hn, w1_ref[...], preferred_element_type=jnp.float32) + fb1_ref[...],
        0.0)
    f = jnp.dot(f, w2_ref[...], preferred_element_type=jnp.float32) + fb2_ref[...]
    z = hn + f
    mu2 = jnp.mean(z, axis=1, keepdims=True)
    var2 = jnp.mean((z - mu2) ** 2, axis=1, keepdims=True)
    o_ref[0] = (z - mu2) / jnp.sqrt(var2 + 1e-5) * g2_ref[...] + b2_ref[...]


def _layer(h, qw, kw, vw, ow, qb, kb, vb, ob, g1, b1, w1, fb1, w2, fb2, g2, b2):
    mat = pl.BlockSpec((D, D), lambda b: (0, 0))
    row = pl.BlockSpec((1, D), lambda b: (0, 0))
    return pl.pallas_call(
        _layer_body,
        grid=(B,),
        in_specs=[
            pl.BlockSpec((1, L, D), lambda b: (b, 0, 0)),
            mat, mat, mat, mat,
            row, row, row, row, row, row,
            pl.BlockSpec((D, FF), lambda b: (0, 0)),
            pl.BlockSpec((1, FF), lambda b: (0, 0)),
            pl.BlockSpec((FF, D), lambda b: (0, 0)),
            row, row, row,
        ],
        out_specs=pl.BlockSpec((1, L, D), lambda b: (b, 0, 0)),
        out_shape=jax.ShapeDtypeStruct((B, L, D), jnp.float32),
    )(h, qw, kw, vw, ow, qb, kb, vb, ob, g1, b1, w1, fb1, w2, fb2, g2, b2)


def _final_body(h_ref, w_ref, b_ref, o_ref):
    pooled = jnp.mean(h_ref[0], axis=0, keepdims=True)
    o_ref[0] = jnp.dot(pooled, w_ref[...], preferred_element_type=jnp.float32) + b_ref[...]


def _final(h, fc_w, fc_b):
    return pl.pallas_call(
        _final_body,
        grid=(B,),
        in_specs=[
            pl.BlockSpec((1, L, D), lambda b: (b, 0, 0)),
            pl.BlockSpec((D, HOR), lambda b: (0, 0)),
            pl.BlockSpec((1, HOR), lambda b: (0, 0)),
        ],
        out_specs=pl.BlockSpec((1, 1, HOR), lambda b: (b, 0, 0)),
        out_shape=jax.ShapeDtypeStruct((B, 1, HOR), jnp.float32),
    )(h, fc_w, fc_b.reshape(1, HOR))


def kernel(x, emb_w, emb_b, q_w, q_b, k_w, k_b, v_w, v_b, o_w, o_b,
           ff1_w, ff1_b, ff2_w, ff2_b, n1_g, n1_b, n2_g, n2_b, fc_w, fc_b):
    h = _embed(x, emb_w, emb_b)
    for i in range(NL):
        h = _layer(h, q_w[i], k_w[i], v_w[i], o_w[i],
                   q_b[i].reshape(1, D), k_b[i].reshape(1, D),
                   v_b[i].reshape(1, D), o_b[i].reshape(1, D),
                   n1_g[i].reshape(1, D), n1_b[i].reshape(1, D),
                   ff1_w[i], ff1_b[i].reshape(1, FF),
                   ff2_w[i], ff2_b[i].reshape(1, D),
                   n2_g[i].reshape(1, D), n2_b[i].reshape(1, D))
    out = _final(h, fc_w, fc_b)
    return out.reshape(B, HOR)


# speedup vs baseline: 2.0290x; 1.5394x over previous
import math

import numpy as np
import jax
import jax.numpy as jnp
from jax.experimental import pallas as pl

B = 2
P = 2048
D = 1024
H = 16
DK = 64
FF = 32
HOR = 24
NL = 2
L = D
U = 6
SCALE = float(DK ** 0.5)


def _pe_np():
    pe = np.zeros((L, D), np.float32)
    pos = np.arange(L, dtype=np.float32)[:, None]
    div = np.exp(np.arange(0, D, 2, dtype=np.float32) * (-math.log(10000.0) / D))
    pe[:, 0::2] = np.sin(pos * div)
    pe[:, 1::2] = np.cos(pos * div)
    return pe


_PE = _pe_np()


def _mask_np():
    m = np.zeros((H * 8, D), np.float32)
    for hh in range(H):
        m[hh * 8:hh * 8 + 8, hh * DK:(hh + 1) * DK] = 1.0
    return m


_MASK = _mask_np()

_RB = 256


def _embed_body(x_ref, w_ref, b_ref, pe_ref, o_ref):
    xb = x_ref[0]
    wb = w_ref[...]
    acc = jax.lax.dot_general(wb, xb, (((0,), (0,)), ((), ())),
                              preferred_element_type=jnp.float32)
    o_ref[0] = acc + b_ref[...] + pe_ref[...]


def _embed(x, emb_w, emb_b):
    return pl.pallas_call(
        _embed_body,
        grid=(B, L // _RB),
        in_specs=[
            pl.BlockSpec((1, P, D), lambda b, j: (b, 0, 0)),
            pl.BlockSpec((P, _RB), lambda b, j: (0, j)),
            pl.BlockSpec((_RB, 1), lambda b, j: (j, 0)),
            pl.BlockSpec((_RB, D), lambda b, j: (j, 0)),
        ],
        out_specs=pl.BlockSpec((1, _RB, D), lambda b, j: (b, j, 0)),
        out_shape=jax.ShapeDtypeStruct((B, L, D), jnp.float32),
    )(x, emb_w, emb_b.reshape(D, 1), jnp.asarray(_PE))


def _layer_body(h_ref, qw_ref, kw_ref, vw_ref, ow_ref, qb_ref, kb_ref, vb_ref,
                ob_ref, g1_ref, b1_ref, w1_ref, fb1_ref, w2_ref, fb2_ref,
                g2_ref, b2_ref, mask_ref, o_ref):
    hb = h_ref[0]
    q_all = jnp.dot(hb, qw_ref[...], preferred_element_type=jnp.float32) + qb_ref[...]
    k_all = jnp.dot(hb, kw_ref[...], preferred_element_type=jnp.float32) + kb_ref[...]
    io_row = jax.lax.broadcasted_iota(jnp.int32, (1, L), 1)
    linb = (jax.lax.broadcasted_iota(jnp.int32, (1, 8, 128), 1) * 128
            + jax.lax.broadcasted_iota(jnp.int32, (1, 8, 128), 2))
    m_list = []
    for hh in range(H):
        sl = slice(hh * DK, (hh + 1) * DK)
        s = jax.lax.dot_general(q_all[:, sl], k_all[:, sl],
                                (((1,), (1,)), ((), ())),
                                preferred_element_type=jnp.float32)
        m_list.append(jnp.max(s.reshape(8, 128, L), axis=-1))
    m3 = jnp.stack(m_list)
    idx_cols = []
    for _ in range(U):
        vmax = jnp.max(m3, axis=(1, 2), keepdims=True)
        idx3 = jnp.min(jnp.where(m3 == vmax, linb, jnp.int32(L)),
                       axis=(1, 2), keepdims=True)
        idx_cols.append(idx3.reshape(H, 1))
        m3 = jnp.where(linb == idx3, jnp.float32(-jnp.inf), m3)
    pad = jnp.full((H, 1), jnp.int32(L), jnp.int32)
    idx_mat = jnp.concatenate(idx_cols + [pad, pad], axis=1)
    e_all = (idx_mat[:, :, None] == io_row.reshape(1, 1, L)
             ).astype(jnp.float32).reshape(H * 8, L)
    maskc = mask_ref[...]
    q_sel = jnp.dot(e_all, q_all, preferred_element_type=jnp.float32) * maskc
    a = jax.lax.dot_general(q_sel, k_all, (((1,), (1,)), ((), ())),
                            preferred_element_type=jnp.float32) / SCALE
    a = a - jnp.max(a, axis=1, keepdims=True)
    w = jnp.exp(a)
    w = w / jnp.sum(w, axis=1, keepdims=True)
    wh = jnp.dot(w, hb, preferred_element_type=jnp.float32)
    ctx = (jnp.dot(wh, vw_ref[...], preferred_element_type=jnp.float32)
           + vb_ref[...]) * maskc
    cf = jnp.dot(ctx, ow_ref[...], preferred_element_type=jnp.float32)
    delta = jax.lax.dot_general(e_all, cf, (((0,), (0,)), ((), ())),
                                preferred_element_type=jnp.float32)
    y = hb + delta + ob_ref[...]
    mu = jnp.mean(y, axis=1, keepdims=True)
    var = jnp.mean((y - mu) ** 2, axis=1, keepdims=True)
    hn = (y - mu) / jnp.sqrt(var + 1e-5) * g1_ref[...] + b1_ref[...]
    f = jnp.maximum(
        jnp.dot(hn, w1_ref[...], preferred_element_type=jnp.float32) + fb1_ref[...],
        0.0)
    f = jnp.dot(f, w2_ref[...], preferred_element_type=jnp.float32) + fb2_ref[...]
    z = hn + f
    mu2 = jnp.mean(z, axis=1, keepdims=True)
    var2 = jnp.mean((z - mu2) ** 2, axis=1, keepdims=True)
    o_ref[0] = (z - mu2) / jnp.sqrt(var2 + 1e-5) * g2_ref[...] + b2_ref[...]


def _layer(h, qw, kw, vw, ow, qb, kb, vb, ob, g1, b1, w1, fb1, w2, fb2, g2, b2):
    mat = pl.BlockSpec((D, D), lambda b: (0, 0))
    row = pl.BlockSpec((1, D), lambda b: (0, 0))
    return pl.pallas_call(
        _layer_body,
        grid=(B,),
        in_specs=[
            pl.BlockSpec((1, L, D), lambda b: (b, 0, 0)),
            mat, mat, mat, mat,
            row, row, row, row, row, row,
            pl.BlockSpec((D, FF), lambda b: (0, 0)),
            pl.BlockSpec((1, FF), lambda b: (0, 0)),
            pl.BlockSpec((FF, D), lambda b: (0, 0)),
            row, row, row,
            pl.BlockSpec((H * 8, D), lambda b: (0, 0)),
        ],
        out_specs=pl.BlockSpec((1, L, D), lambda b: (b, 0, 0)),
        out_shape=jax.ShapeDtypeStruct((B, L, D), jnp.float32),
    )(h, qw, kw, vw, ow, qb, kb, vb, ob, g1, b1, w1, fb1, w2, fb2, g2, b2,
      jnp.asarray(_MASK))


def _final_body(h_ref, w_ref, b_ref, o_ref):
    pooled = jnp.mean(h_ref[0], axis=0, keepdims=True)
    o_ref[0] = jnp.dot(pooled, w_ref[...], preferred_element_type=jnp.float32) + b_ref[...]


def _final(h, fc_w, fc_b):
    return pl.pallas_call(
        _final_body,
        grid=(B,),
        in_specs=[
            pl.BlockSpec((1, L, D), lambda b: (b, 0, 0)),
            pl.BlockSpec((D, HOR), lambda b: (0, 0)),
            pl.BlockSpec((1, HOR), lambda b: (0, 0)),
        ],
        out_specs=pl.BlockSpec((1, 1, HOR), lambda b: (b, 0, 0)),
        out_shape=jax.ShapeDtypeStruct((B, 1, HOR), jnp.float32),
    )(h, fc_w, fc_b.reshape(1, HOR))


def kernel(x, emb_w, emb_b, q_w, q_b, k_w, k_b, v_w, v_b, o_w, o_b,
           ff1_w, ff1_b, ff2_w, ff2_b, n1_g, n1_b, n2_g, n2_b, fc_w, fc_b):
    h = _embed(x, emb_w, emb_b)
    for i in range(NL):
        h = _layer(h, q_w[i], k_w[i], v_w[i], o_w[i],
                   q_b[i].reshape(1, D), k_b[i].reshape(1, D),
                   v_b[i].reshape(1, D), o_b[i].reshape(1, D),
                   n1_g[i].reshape(1, D), n1_b[i].reshape(1, D),
                   ff1_w[i], ff1_b[i].reshape(1, FF),
                   ff2_w[i], ff2_b[i].reshape(1, D),
                   n2_g[i].reshape(1, D), n2_b[i].reshape(1, D))
    out = _final(h, fc_w, fc_b)
    return out.reshape(B, HOR)


# speedup vs baseline: 2.0579x; 1.0142x over previous
import math

import numpy as np
import jax
import jax.numpy as jnp
from jax.experimental import pallas as pl

B = 2
P = 2048
D = 1024
H = 16
DK = 64
FF = 32
HOR = 24
NL = 2
L = D
U = 6
SCALE = float(DK ** 0.5)


def _pe_np():
    pe = np.zeros((L, D), np.float32)
    pos = np.arange(L, dtype=np.float32)[:, None]
    div = np.exp(np.arange(0, D, 2, dtype=np.float32) * (-math.log(10000.0) / D))
    pe[:, 0::2] = np.sin(pos * div)
    pe[:, 1::2] = np.cos(pos * div)
    return pe


_PE = _pe_np()


def _mask_np():
    m = np.zeros((H * 8, D), np.float32)
    for hh in range(H):
        m[hh * 8:hh * 8 + 8, hh * DK:(hh + 1) * DK] = 1.0
    return m


_MASK = _mask_np()

_RB = 256


def _embed_body(x_ref, w_ref, b_ref, pe_ref, o_ref):
    xb = x_ref[0]
    wb = w_ref[...]
    acc = jax.lax.dot_general(wb, xb, (((0,), (0,)), ((), ())),
                              preferred_element_type=jnp.float32)
    o_ref[0] = acc + b_ref[...] + pe_ref[...]


def _embed(x, emb_w, emb_b):
    return pl.pallas_call(
        _embed_body,
        grid=(B, L // _RB),
        in_specs=[
            pl.BlockSpec((1, P, D), lambda b, j: (b, 0, 0)),
            pl.BlockSpec((P, _RB), lambda b, j: (0, j)),
            pl.BlockSpec((_RB, 1), lambda b, j: (j, 0)),
            pl.BlockSpec((_RB, D), lambda b, j: (j, 0)),
        ],
        out_specs=pl.BlockSpec((1, _RB, D), lambda b, j: (b, j, 0)),
        out_shape=jax.ShapeDtypeStruct((B, L, D), jnp.float32),
    )(x, emb_w, emb_b.reshape(D, 1), jnp.asarray(_PE))


def _layer_body(h_ref, qw_ref, kw_ref, vw_ref, ow_ref, qb_ref, kb_ref, vb_ref,
                ob_ref, g1_ref, b1_ref, w1_ref, fb1_ref, w2_ref, fb2_ref,
                g2_ref, b2_ref, mask_ref, o_ref):
    hb = h_ref[0]
    hb16 = hb.astype(jnp.bfloat16)
    q_all = jnp.dot(hb16, qw_ref[...].astype(jnp.bfloat16),
                    preferred_element_type=jnp.float32) + qb_ref[...]
    k_all = jnp.dot(hb16, kw_ref[...].astype(jnp.bfloat16),
                    preferred_element_type=jnp.float32) + kb_ref[...]
    q16 = q_all.astype(jnp.bfloat16)
    k16 = k_all.astype(jnp.bfloat16)
    io_row = jax.lax.broadcasted_iota(jnp.int32, (1, L), 1)
    linb = (jax.lax.broadcasted_iota(jnp.int32, (1, 8, 128), 1) * 128
            + jax.lax.broadcasted_iota(jnp.int32, (1, 8, 128), 2))
    m_list = []
    for hh in range(H):
        sl = slice(hh * DK, (hh + 1) * DK)
        s = jax.lax.dot_general(q16[:, sl], k16[:, sl],
                                (((1,), (1,)), ((), ())),
                                preferred_element_type=jnp.float32)
        m_list.append(jnp.max(s.reshape(8, 128, L), axis=-1))
    m3 = jnp.stack(m_list)
    idx_cols = []
    for _ in range(U):
        vmax = jnp.max(m3, axis=(1, 2), keepdims=True)
        idx3 = jnp.min(jnp.where(m3 == vmax, linb, jnp.int32(L)),
                       axis=(1, 2), keepdims=True)
        idx_cols.append(idx3.reshape(H, 1))
        m3 = jnp.where(linb == idx3, jnp.float32(-jnp.inf), m3)
    pad = jnp.full((H, 1), jnp.int32(L), jnp.int32)
    idx_mat = jnp.concatenate(idx_cols + [pad, pad], axis=1)
    e_all = (idx_mat[:, :, None] == io_row.reshape(1, 1, L)
             ).astype(jnp.float32).reshape(H * 8, L)
    maskc = mask_ref[...]
    q_sel = jnp.dot(e_all, q_all, preferred_element_type=jnp.float32) * maskc
    a = jax.lax.dot_general(q_sel, k_all, (((1,), (1,)), ((), ())),
                            preferred_element_type=jnp.float32) / SCALE
    a = a - jnp.max(a, axis=1, keepdims=True)
    w = jnp.exp(a)
    w = w / jnp.sum(w, axis=1, keepdims=True)
    wh = jnp.dot(w, hb, preferred_element_type=jnp.float32)
    ctx = (jnp.dot(wh, vw_ref[...], preferred_element_type=jnp.float32)
           + vb_ref[...]) * maskc
    cf = jnp.dot(ctx, ow_ref[...], preferred_element_type=jnp.float32)
    delta = jax.lax.dot_general(e_all, cf, (((0,), (0,)), ((), ())),
                                preferred_element_type=jnp.float32)
    y = hb + delta + ob_ref[...]
    mu = jnp.mean(y, axis=1, keepdims=True)
    var = jnp.mean((y - mu) ** 2, axis=1, keepdims=True)
    hn = (y - mu) / jnp.sqrt(var + 1e-5) * g1_ref[...] + b1_ref[...]
    f = jnp.maximum(
        jnp.dot(hn, w1_ref[...], preferred_element_type=jnp.float32) + fb1_ref[...],
        0.0)
    f = jnp.dot(f, w2_ref[...], preferred_element_type=jnp.float32) + fb2_ref[...]
    z = hn + f
    mu2 = jnp.mean(z, axis=1, keepdims=True)
    var2 = jnp.mean((z - mu2) ** 2, axis=1, keepdims=True)
    o_ref[0] = (z - mu2) / jnp.sqrt(var2 + 1e-5) * g2_ref[...] + b2_ref[...]


def _layer(h, qw, kw, vw, ow, qb, kb, vb, ob, g1, b1, w1, fb1, w2, fb2, g2, b2):
    mat = pl.BlockSpec((D, D), lambda b: (0, 0))
    row = pl.BlockSpec((1, D), lambda b: (0, 0))
    return pl.pallas_call(
        _layer_body,
        grid=(B,),
        in_specs=[
            pl.BlockSpec((1, L, D), lambda b: (b, 0, 0)),
            mat, mat, mat, mat,
            row, row, row, row, row, row,
            pl.BlockSpec((D, FF), lambda b: (0, 0)),
            pl.BlockSpec((1, FF), lambda b: (0, 0)),
            pl.BlockSpec((FF, D), lambda b: (0, 0)),
            row, row, row,
            pl.BlockSpec((H * 8, D), lambda b: (0, 0)),
        ],
        out_specs=pl.BlockSpec((1, L, D), lambda b: (b, 0, 0)),
        out_shape=jax.ShapeDtypeStruct((B, L, D), jnp.float32),
    )(h, qw, kw, vw, ow, qb, kb, vb, ob, g1, b1, w1, fb1, w2, fb2, g2, b2,
      jnp.asarray(_MASK))


def _final_body(h_ref, w_ref, b_ref, o_ref):
    pooled = jnp.mean(h_ref[0], axis=0, keepdims=True)
    o_ref[0] = jnp.dot(pooled, w_ref[...], preferred_element_type=jnp.float32) + b_ref[...]


def _final(h, fc_w, fc_b):
    return pl.pallas_call(
        _final_body,
        grid=(B,),
        in_specs=[
            pl.BlockSpec((1, L, D), lambda b: (b, 0, 0)),
            pl.BlockSpec((D, HOR), lambda b: (0, 0)),
            pl.BlockSpec((1, HOR), lambda b: (0, 0)),
        ],
        out_specs=pl.BlockSpec((1, 1, HOR), lambda b: (b, 0, 0)),
        out_shape=jax.ShapeDtypeStruct((B, 1, HOR), jnp.float32),
    )(h, fc_w, fc_b.reshape(1, HOR))


def kernel(x, emb_w, emb_b, q_w, q_b, k_w, k_b, v_w, v_b, o_w, o_b,
           ff1_w, ff1_b, ff2_w, ff2_b, n1_g, n1_b, n2_g, n2_b, fc_w, fc_b):
    h = _embed(x, emb_w, emb_b)
    for i in range(NL):
        h = _layer(h, q_w[i], k_w[i], v_w[i], o_w[i],
                   q_b[i].reshape(1, D), k_b[i].reshape(1, D),
                   v_b[i].reshape(1, D), o_b[i].reshape(1, D),
                   n1_g[i].reshape(1, D), n1_b[i].reshape(1, D),
                   ff1_w[i], ff1_b[i].reshape(1, FF),
                   ff2_w[i], ff2_b[i].reshape(1, D),
                   n2_g[i].reshape(1, D), n2_b[i].reshape(1, D))
    out = _final(h, fc_w, fc_b)
    return out.reshape(B, HOR)


# speedup vs baseline: 2.0989x; 1.0199x over previous
import math

import numpy as np
import jax
import jax.numpy as jnp
from jax.experimental import pallas as pl

B = 2
P = 2048
D = 1024
H = 16
DK = 64
FF = 32
HOR = 24
NL = 2
L = D
U = 6
SCALE = float(DK ** 0.5)


def _pe_np():
    pe = np.zeros((L, D), np.float32)
    pos = np.arange(L, dtype=np.float32)[:, None]
    div = np.exp(np.arange(0, D, 2, dtype=np.float32) * (-math.log(10000.0) / D))
    pe[:, 0::2] = np.sin(pos * div)
    pe[:, 1::2] = np.cos(pos * div)
    return pe


_PE = _pe_np()

_RB = 256


def _embed_body(x_ref, w_ref, b_ref, pe_ref, o_ref):
    xb = x_ref[0]
    wb = w_ref[...]
    acc = jax.lax.dot_general(wb, xb, (((0,), (0,)), ((), ())),
                              preferred_element_type=jnp.float32)
    o_ref[0] = acc + b_ref[...] + pe_ref[...]


def _embed(x, emb_w, emb_b):
    return pl.pallas_call(
        _embed_body,
        grid=(B, L // _RB),
        in_specs=[
            pl.BlockSpec((1, P, D), lambda b, j: (b, 0, 0)),
            pl.BlockSpec((P, _RB), lambda b, j: (0, j)),
            pl.BlockSpec((_RB, 1), lambda b, j: (j, 0)),
            pl.BlockSpec((_RB, D), lambda b, j: (j, 0)),
        ],
        out_specs=pl.BlockSpec((1, _RB, D), lambda b, j: (b, j, 0)),
        out_shape=jax.ShapeDtypeStruct((B, L, D), jnp.float32),
    )(x, emb_w, emb_b.reshape(D, 1), jnp.asarray(_PE))


def _layer_body(h_ref, qw_ref, kw_ref, vw_ref, ow_ref, qb_ref, kb_ref, vb_ref,
                ob_ref, g1_ref, b1_ref, w1_ref, fb1_ref, w2_ref, fb2_ref,
                g2_ref, b2_ref, o_ref):
    hb = h_ref[0]
    hb16 = hb.astype(jnp.bfloat16)
    q_all = jnp.dot(hb16, qw_ref[...],
                    preferred_element_type=jnp.float32) + qb_ref[...]
    k_all = jnp.dot(hb16, kw_ref[...],
                    preferred_element_type=jnp.float32) + kb_ref[...]
    q16 = q_all.astype(jnp.bfloat16)
    k16 = k_all.astype(jnp.bfloat16)
    io_row = jax.lax.broadcasted_iota(jnp.int32, (1, L), 1)
    linb = (jax.lax.broadcasted_iota(jnp.int32, (1, 8, 128), 1) * 128
            + jax.lax.broadcasted_iota(jnp.int32, (1, 8, 128), 2))
    m_list = []
    for hh in range(H):
        sl = slice(hh * DK, (hh + 1) * DK)
        s = jax.lax.dot_general(q16[:, sl], k16[:, sl],
                                (((1,), (1,)), ((), ())),
                                preferred_element_type=jnp.float32)
        m_list.append(jnp.max(s.reshape(8, 128, L), axis=-1))
    m3 = jnp.stack(m_list)
    idx_cols = []
    for _ in range(U):
        vmax = jnp.max(m3, axis=(1, 2), keepdims=True)
        idx3 = jnp.min(jnp.where(m3 == vmax, linb, jnp.int32(L)),
                       axis=(1, 2), keepdims=True)
        idx_cols.append(idx3.reshape(H, 1))
        m3 = jnp.where(linb == idx3, jnp.float32(-jnp.inf), m3)
    pad = jnp.full((H, 1), jnp.int32(L), jnp.int32)
    idx_mat = jnp.concatenate(idx_cols + [pad, pad], axis=1)
    e_all = (idx_mat[:, :, None] == io_row.reshape(1, 1, L)
             ).astype(jnp.float32).reshape(H * 8, L)
    rg = jax.lax.broadcasted_iota(jnp.int32, (H * 8, 1), 0) // 8
    maskc = (rg == io_row // DK).astype(jnp.float32)
    q_sel = jnp.dot(e_all, q_all, preferred_element_type=jnp.float32) * maskc
    a = jax.lax.dot_general(q_sel, k_all, (((1,), (1,)), ((), ())),
                            preferred_element_type=jnp.float32) / SCALE
    a = a - jnp.max(a, axis=1, keepdims=True)
    w = jnp.exp(a)
    w = w / jnp.sum(w, axis=1, keepdims=True)
    wh = jnp.dot(w, hb, preferred_element_type=jnp.float32)
    ctx = (jnp.dot(wh, vw_ref[...], preferred_element_type=jnp.float32)
           + vb_ref[...]) * maskc
    cf = jnp.dot(ctx, ow_ref[...], preferred_element_type=jnp.float32)
    delta = jax.lax.dot_general(e_all, cf, (((0,), (0,)), ((), ())),
                                preferred_element_type=jnp.float32)
    y = hb + delta + ob_ref[...]
    mu = jnp.mean(y, axis=1, keepdims=True)
    var = jnp.mean((y - mu) ** 2, axis=1, keepdims=True)
    hn = (y - mu) / jnp.sqrt(var + 1e-5) * g1_ref[...] + b1_ref[...]
    f = jnp.maximum(
        jnp.dot(hn, w1_ref[...], preferred_element_type=jnp.float32) + fb1_ref[...],
        0.0)
    f = jnp.dot(f, w2_ref[...], preferred_element_type=jnp.float32) + fb2_ref[...]
    z = hn + f
    mu2 = jnp.mean(z, axis=1, keepdims=True)
    var2 = jnp.mean((z - mu2) ** 2, axis=1, keepdims=True)
    o_ref[0] = (z - mu2) / jnp.sqrt(var2 + 1e-5) * g2_ref[...] + b2_ref[...]


def _layer(h, qw, kw, vw, ow, qb, kb, vb, ob, g1, b1, w1, fb1, w2, fb2, g2, b2):
    mat = pl.BlockSpec((D, D), lambda b: (0, 0))
    row = pl.BlockSpec((1, D), lambda b: (0, 0))
    return pl.pallas_call(
        _layer_body,
        grid=(B,),
        in_specs=[
            pl.BlockSpec((1, L, D), lambda b: (b, 0, 0)),
            mat, mat, mat, mat,
            row, row, row, row, row, row,
            pl.BlockSpec((D, FF), lambda b: (0, 0)),
            pl.BlockSpec((1, FF), lambda b: (0, 0)),
            pl.BlockSpec((FF, D), lambda b: (0, 0)),
            row, row, row,
        ],
        out_specs=pl.BlockSpec((1, L, D), lambda b: (b, 0, 0)),
        out_shape=jax.ShapeDtypeStruct((B, L, D), jnp.float32),
    )(h, qw, kw, vw, ow, qb, kb, vb, ob, g1, b1, w1, fb1, w2, fb2, g2, b2)


def _final_body(h_ref, w_ref, b_ref, o_ref):
    pooled = jnp.mean(h_ref[0], axis=0, keepdims=True)
    o_ref[0] = jnp.dot(pooled, w_ref[...], preferred_element_type=jnp.float32) + b_ref[...]


def _final(h, fc_w, fc_b):
    return pl.pallas_call(
        _final_body,
        grid=(B,),
        in_specs=[
            pl.BlockSpec((1, L, D), lambda b: (b, 0, 0)),
            pl.BlockSpec((D, HOR), lambda b: (0, 0)),
            pl.BlockSpec((1, HOR), lambda b: (0, 0)),
        ],
        out_specs=pl.BlockSpec((1, 1, HOR), lambda b: (b, 0, 0)),
        out_shape=jax.ShapeDtypeStruct((B, 1, HOR), jnp.float32),
    )(h, fc_w, fc_b.reshape(1, HOR))


def kernel(x, emb_w, emb_b, q_w, q_b, k_w, k_b, v_w, v_b, o_w, o_b,
           ff1_w, ff1_b, ff2_w, ff2_b, n1_g, n1_b, n2_g, n2_b, fc_w, fc_b):
    h = _embed(x, emb_w, emb_b)
    for i in range(NL):
        h = _layer(h, q_w[i].astype(jnp.bfloat16), k_w[i].astype(jnp.bfloat16),
                   v_w[i], o_w[i],
                   q_b[i].reshape(1, D), k_b[i].reshape(1, D),
                   v_b[i].reshape(1, D), o_b[i].reshape(1, D),
                   n1_g[i].reshape(1, D), n1_b[i].reshape(1, D),
                   ff1_w[i], ff1_b[i].reshape(1, FF),
                   ff2_w[i], ff2_b[i].reshape(1, D),
                   n2_g[i].reshape(1, D), n2_b[i].reshape(1, D))
    out = _final(h, fc_w, fc_b)
    return out.reshape(B, HOR)


# speedup vs baseline: 2.3968x; 1.1420x over previous
import math

import numpy as np
import jax
import jax.numpy as jnp
from jax.experimental import pallas as pl
from jax.experimental.pallas import tpu as pltpu

B = 2
P = 2048
D = 1024
H = 16
DK = 64
FF = 32
HOR = 24
NL = 2
L = D
U = 6
SCALE = float(DK ** 0.5)


def _pe_np():
    pe = np.zeros((L, D), np.float32)
    pos = np.arange(L, dtype=np.float32)[:, None]
    div = np.exp(np.arange(0, D, 2, dtype=np.float32) * (-math.log(10000.0) / D))
    pe[:, 0::2] = np.sin(pos * div)
    pe[:, 1::2] = np.cos(pos * div)
    return pe


_PE = _pe_np()

_RB = 256


def _embed_body(x_ref, w_ref, b_ref, pe_ref, o_ref):
    xb = x_ref[0]
    wb = w_ref[...]
    acc = jax.lax.dot_general(wb, xb, (((0,), (0,)), ((), ())),
                              preferred_element_type=jnp.float32)
    o_ref[0] = acc + b_ref[...] + pe_ref[...]


def _embed(x, emb_w, emb_b):
    return pl.pallas_call(
        _embed_body,
        grid=(B, L // _RB),
        in_specs=[
            pl.BlockSpec((1, P, D), lambda b, j: (b, 0, 0)),
            pl.BlockSpec((P, _RB), lambda b, j: (0, j)),
            pl.BlockSpec((_RB, 1), lambda b, j: (j, 0)),
            pl.BlockSpec((_RB, D), lambda b, j: (j, 0)),
        ],
        out_specs=pl.BlockSpec((1, _RB, D), lambda b, j: (b, j, 0)),
        out_shape=jax.ShapeDtypeStruct((B, L, D), jnp.float32),
    )(x, emb_w, emb_b.reshape(D, 1), jnp.asarray(_PE))


def _net_body(h0_ref, qw_ref, kw_ref, vw_ref, ow_ref, qb_ref, kb_ref, vb_ref,
              ob_ref, g1_ref, b1_ref, w1_ref, fb1_ref, w2_ref, fb2_ref,
              g2_ref, b2_ref, fcw_ref, fcb_ref, o_ref, hs_ref):
    i = pl.program_id(1)

    @pl.when(i == 0)
    def _():
        hs_ref[...] = h0_ref[0]

    hb = hs_ref[...]
    hb16 = hb.astype(jnp.bfloat16)
    q_all = jnp.dot(hb16, qw_ref[0],
                    preferred_element_type=jnp.float32) + qb_ref[0]
    k_all = jnp.dot(hb16, kw_ref[0],
                    preferred_element_type=jnp.float32) + kb_ref[0]
    q16 = q_all.astype(jnp.bfloat16)
    k16 = k_all.astype(jnp.bfloat16)
    io_row = jax.lax.broadcasted_iota(jnp.int32, (1, L), 1)
    linb = (jax.lax.broadcasted_iota(jnp.int32, (1, 8, 128), 1) * 128
            + jax.lax.broadcasted_iota(jnp.int32, (1, 8, 128), 2))
    m_list = []
    for hh in range(H):
        sl = slice(hh * DK, (hh + 1) * DK)
        s = jax.lax.dot_general(q16[:, sl], k16[:, sl],
                                (((1,), (1,)), ((), ())),
                                preferred_element_type=jnp.float32)
        m_list.append(jnp.max(s.reshape(8, 128, L), axis=-1))
    m3 = jnp.stack(m_list)
    idx_cols = []
    for _ in range(U):
        vmax = jnp.max(m3, axis=(1, 2), keepdims=True)
        idx3 = jnp.min(jnp.where(m3 == vmax, linb, jnp.int32(L)),
                       axis=(1, 2), keepdims=True)
        idx_cols.append(idx3.reshape(H, 1))
        m3 = jnp.where(linb == idx3, jnp.float32(-jnp.inf), m3)
    pad = jnp.full((H, 1), jnp.int32(L), jnp.int32)
    idx_mat = jnp.concatenate(idx_cols + [pad, pad], axis=1)
    e_all = (idx_mat[:, :, None] == io_row.reshape(1, 1, L)
             ).astype(jnp.float32).reshape(H * 8, L)
    rg = jax.lax.broadcasted_iota(jnp.int32, (H * 8, 1), 0) // 8
    maskc = (rg == io_row // DK).astype(jnp.float32)
    q_sel = jnp.dot(e_all, q_all, preferred_element_type=jnp.float32) * maskc
    a = jax.lax.dot_general(q_sel, k_all, (((1,), (1,)), ((), ())),
                            preferred_element_type=jnp.float32) / SCALE
    a = a - jnp.max(a, axis=1, keepdims=True)
    w = jnp.exp(a)
    w = w / jnp.sum(w, axis=1, keepdims=True)
    wh = jnp.dot(w, hb, preferred_element_type=jnp.float32)
    ctx = (jnp.dot(wh.astype(jnp.bfloat16), vw_ref[0],
                   preferred_element_type=jnp.float32)
           + vb_ref[0]) * maskc
    cf = jnp.dot(ctx.astype(jnp.bfloat16), ow_ref[0],
                 preferred_element_type=jnp.float32)
    delta = jax.lax.dot_general(e_all, cf, (((0,), (0,)), ((), ())),
                                preferred_element_type=jnp.float32)
    y = hb + delta + ob_ref[0]
    mu = jnp.mean(y, axis=1, keepdims=True)
    var = jnp.mean((y - mu) ** 2, axis=1, keepdims=True)
    hn = (y - mu) / jnp.sqrt(var + 1e-5) * g1_ref[0] + b1_ref[0]
    f = jnp.maximum(
        jnp.dot(hn, w1_ref[0], preferred_element_type=jnp.float32) + fb1_ref[0],
        0.0)
    f = jnp.dot(f, w2_ref[0], preferred_element_type=jnp.float32) + fb2_ref[0]
    z = hn + f
    mu2 = jnp.mean(z, axis=1, keepdims=True)
    var2 = jnp.mean((z - mu2) ** 2, axis=1, keepdims=True)
    zf = (z - mu2) / jnp.sqrt(var2 + 1e-5) * g2_ref[0] + b2_ref[0]
    hs_ref[...] = zf
    pooled = jnp.mean(zf, axis=0, keepdims=True)
    o_ref[0] = (jnp.dot(pooled, fcw_ref[...], preferred_element_type=jnp.float32)
                + fcb_ref[...])


def _net(h0, qw, kw, vw, ow, qb, kb, vb, ob, g1, b1, w1, fb1, w2, fb2, g2, b2,
         fc_w, fc_b):
    mat = pl.BlockSpec((1, D, D), lambda b, i: (i, 0, 0))
    row = pl.BlockSpec((1, 1, D), lambda b, i: (i, 0, 0))
    return pl.pallas_call(
        _net_body,
        grid=(B, NL),
        in_specs=[
            pl.BlockSpec((1, L, D), lambda b, i: (b, 0, 0)),
            mat, mat, mat, mat,
            row, row, row, row, row, row,
            pl.BlockSpec((1, D, FF), lambda b, i: (i, 0, 0)),
            pl.BlockSpec((1, 1, FF), lambda b, i: (i, 0, 0)),
            pl.BlockSpec((1, FF, D), lambda b, i: (i, 0, 0)),
            row, row, row,
            pl.BlockSpec((D, HOR), lambda b, i: (0, 0)),
            pl.BlockSpec((1, HOR), lambda b, i: (0, 0)),
        ],
        out_specs=pl.BlockSpec((1, 1, HOR), lambda b, i: (b, 0, 0)),
        out_shape=jax.ShapeDtypeStruct((B, 1, HOR), jnp.float32),
        scratch_shapes=[pltpu.VMEM((L, D), jnp.float32)],
    )(h0, qw, kw, vw, ow, qb, kb, vb, ob, g1, b1, w1, fb1, w2, fb2, g2, b2,
      fc_w, fc_b.reshape(1, HOR))


def kernel(x, emb_w, emb_b, q_w, q_b, k_w, k_b, v_w, v_b, o_w, o_b,
           ff1_w, ff1_b, ff2_w, ff2_b, n1_g, n1_b, n2_g, n2_b, fc_w, fc_b):
    h0 = _embed(x, emb_w, emb_b)
    out = _net(h0,
               q_w.astype(jnp.bfloat16), k_w.astype(jnp.bfloat16),
               v_w.astype(jnp.bfloat16), o_w.astype(jnp.bfloat16),
               q_b.reshape(NL, 1, D), k_b.reshape(NL, 1, D),
               v_b.reshape(NL, 1, D), o_b.reshape(NL, 1, D),
               n1_g.reshape(NL, 1, D), n1_b.reshape(NL, 1, D),
               ff1_w, ff1_b.reshape(NL, 1, FF),
               ff2_w, ff2_b.reshape(NL, 1, D),
               n2_g.reshape(NL, 1, D), n2_b.reshape(NL, 1, D),
               fc_w, fc_b)
    return out.reshape(B, HOR)
